# Initial kernel scaffold; baseline (speedup 1.0000x reference)
import jax
import jax.numpy as jnp
from jax.experimental import pallas as pl


def kernel(x, edge_index, W1, att_l1, att_r1, W2, att_l2, att_r2, Wp1, bp1, Wp2, bp2):
    raise NotImplementedError("write your pallas kernel here")


# speedup vs baseline: 15.6348x; 15.6348x over previous
import jax
import jax.numpy as jnp
from jax import lax
from jax.experimental import pallas as pl
from jax.experimental.pallas import tpu as pltpu
from jax.experimental.pallas import tpu_sc as plsc

N_NODES = 10000
N_EDGES = 320000
D = 128
NEG_SLOPE = 0.2

NC = 2
NS = 16
NW = NC * NS
NB = NS
EPB = N_EDGES // NB
HALF = 5120
NH = NC * HALF
N_PAD = 10112
RPS = HALF // NS
CH = 80
NCH = EPB // CH
L = 16

_f32 = jnp.float32


def _proj(xl, attl_ref, attr_ref):
    dn = (((1,), (0,)), ((), ()))
    al = lax.dot_general(xl, attl_ref[...], dn,
                         preferred_element_type=_f32)
    ar = lax.dot_general(xl, attr_ref[...], dn,
                         preferred_element_type=_f32)
    amax = jnp.broadcast_to(jnp.max(al), (1, D))
    return al, ar, amax


def _tc_first_body(x_ref, w_ref, attl_ref, attr_ref,
                   xl_ref, al_ref, ar_ref, amax_ref):
    xl = lax.dot_general(x_ref[...], w_ref[...], (((1,), (1,)), ((), ())),
                         preferred_element_type=_f32)
    xl_ref[...] = xl
    al, ar, amax = _proj(xl, attl_ref, attr_ref)
    al_ref[...] = al
    ar_ref[...] = ar
    amax_ref[...] = amax


def _combine(acc_ref, s_ref):
    s = lax.dot_general(s_ref[...], jnp.ones((NW, 1), _f32),
                        (((0,), (0,)), ((), ())),
                        preferred_element_type=_f32)[:N_NODES]
    acc = acc_ref[...][:N_NODES]
    safe = jnp.where(s > 0, s, 1.0)
    h = jnp.where(s > 0, acc / safe, 0.0)
    return jnp.maximum(h, 0.0)


def _tc_mid_body(acc_ref, s_ref, w_ref, attl_ref, attr_ref,
                 xl_ref, al_ref, ar_ref, amax_ref):
    h = _combine(acc_ref, s_ref)
    xl = lax.dot_general(h, w_ref[...], (((1,), (1,)), ((), ())),
                         preferred_element_type=_f32)
    xl_ref[...] = xl
    al, ar, amax = _proj(xl, attl_ref, attr_ref)
    al_ref[...] = al
    ar_ref[...] = ar
    amax_ref[...] = amax


def _tc_out_body(acc_ref, s_ref, wp1_ref, bp1_ref, wp2_ref, bp2_ref, out_ref):
    h = _combine(acc_ref, s_ref)
    t = lax.dot_general(h, wp1_ref[...], (((1,), (1,)), ((), ())),
                        preferred_element_type=_f32) + bp1_ref[...][None, :]
    out_ref[...] = lax.dot_general(t, wp2_ref[...], (((1,), (1,)), ((), ())),
                                   preferred_element_type=_f32) + bp2_ref[...][None, :]


_lin_out = (jax.ShapeDtypeStruct((N_NODES, D), _f32),
            jax.ShapeDtypeStruct((N_NODES, 1), _f32),
            jax.ShapeDtypeStruct((N_NODES, 1), _f32),
            jax.ShapeDtypeStruct((1, D), _f32))

_tc_first = pl.pallas_call(_tc_first_body, out_shape=_lin_out)
_tc_mid = pl.pallas_call(_tc_mid_body, out_shape=_lin_out)
_tc_out = pl.pallas_call(
    _tc_out_body, out_shape=jax.ShapeDtypeStruct((N_NODES, D), _f32))


def _edge_body(al_hbm, ar_hbm, amax_hbm, src_hbm, dst_hbm, xl_hbm,
               zrows_hbm, z1d_hbm,
               acc_out, s_out,
               al_v, ar_v, amax_v, s_v, src_v, dst_v, wgt_v, rows_v, didx_v,
               acc_sh, sem):
    c = lax.axis_index("c")
    sid = lax.axis_index("s")
    w = sid * NC + c
    eb = sid * EPB
    lo = c * HALF

    pltpu.sync_copy(zrows_hbm, acc_sh.at[pl.ds(sid * RPS, RPS)])

    @pl.when(sid == 0)
    def _zero_tail():
        pltpu.sync_copy(zrows_hbm.at[pl.ds(0, L)], acc_sh.at[pl.ds(HALF, L)])

    pltpu.sync_copy(al_hbm, al_v)
    pltpu.sync_copy(ar_hbm, ar_v)
    pltpu.sync_copy(amax_hbm, amax_v)
    pltpu.sync_copy(z1d_hbm, s_v)
    pltpu.sync_copy(src_hbm.at[pl.ds(eb, EPB)], src_v)
    pltpu.sync_copy(dst_hbm.at[pl.ds(eb, EPB)], dst_v)
    plsc.subcore_barrier()
    amax = amax_v[0, pl.ds(0, L)]

    def rbody(ci, carry):
        base = pl.multiple_of(ci * CH, CH)
        cp = pltpu.async_copy(xl_hbm.at[src_v.at[pl.ds(base, CH)]], rows_v, sem)
        for q in range(CH // L):
            s16 = src_v[pl.ds(base + q * L, L)]
            d16 = dst_v[pl.ds(base + q * L, L)]
            als = plsc.load_gather(al_v, [s16])
            ard = plsc.load_gather(ar_v, [d16])
            z = als + ard
            a = jnp.where(z > 0, z, NEG_SLOPE * z)
            zz = amax + ard
            md = jnp.where(zz > 0, zz, NEG_SLOPE * zz)
            wv = jnp.exp(a - md)
            owned = (d16 >= lo) & (d16 < lo + HALF)
            wgt_v[pl.ds(q * L, L)] = jnp.where(owned, wv, 0.0)
            plsc.addupdate_scatter(s_v, [d16], wv, mask=owned)
            didx_v[pl.ds(q * L, L)] = jnp.where(owned, d16 - lo, HALF)
        cp.wait()

        def sbody(j, inner):
            wj = plsc.load_gather(wgt_v, [lax.broadcast(j, (L,))])
            for q in range(D // L):
                sl = pl.ds(q * L, L)
                rows_v[j, sl] = rows_v[j, sl] * wj
            return inner

        lax.fori_loop(0, CH, sbody, 0)
        pltpu.sync_copy(rows_v, acc_sh.at[didx_v], add=True)
        return carry

    lax.fori_loop(0, NCH, rbody, 0)

    plsc.subcore_barrier()
    pltpu.sync_copy(acc_sh.at[pl.ds(sid * RPS, RPS)],
                    acc_out.at[pl.ds(c * HALF + sid * RPS, RPS)])
    pltpu.sync_copy(s_v, s_out.at[pl.ds(w * N_PAD, N_PAD)])


_edge = pl.kernel(
    _edge_body,
    out_type=[jax.ShapeDtypeStruct((NH, D), _f32),
              jax.ShapeDtypeStruct((NW * N_PAD,), _f32)],
    mesh=plsc.VectorSubcoreMesh(core_axis_name="c", subcore_axis_name="s",
                                num_cores=NC, num_subcores=NS),
    scratch_types=[
        pltpu.VMEM((N_NODES,), _f32),
        pltpu.VMEM((N_NODES,), _f32),
        pltpu.VMEM((1, D), _f32),
        pltpu.VMEM((N_PAD,), _f32),
        pltpu.VMEM((EPB,), jnp.int32),
        pltpu.VMEM((EPB,), jnp.int32),
        pltpu.VMEM((CH,), _f32),
        pltpu.VMEM((CH, D), _f32),
        pltpu.VMEM((CH,), jnp.int32),
        pltpu.VMEM_SHARED((HALF + L, D), _f32),
        pltpu.SemaphoreType.DMA,
    ],
    compiler_params=pltpu.CompilerParams(needs_layout_passes=False),
)


def kernel(x, edge_index, W1, att_l1, att_r1, W2, att_l2, att_r2,
           Wp1, bp1, Wp2, bp2):
    src = edge_index[0].astype(jnp.int32)
    dst = edge_index[1].astype(jnp.int32)
    attl1 = att_l1.reshape(D, 1)
    attr1 = att_r1.reshape(D, 1)
    attl2 = att_l2.reshape(D, 1)
    attr2 = att_r2.reshape(D, 1)
    zrows = jnp.zeros((RPS, D), _f32)
    z1d = jnp.zeros((N_PAD,), _f32)

    xl1, al1, ar1, amax1 = _tc_first(x, W1, attl1, attr1)
    acc1, s1 = _edge(al1.reshape(N_NODES), ar1.reshape(N_NODES),
                     amax1, src, dst, xl1, zrows, z1d)
    xl2, al2, ar2, amax2 = _tc_mid(acc1, s1.reshape(NW, N_PAD), W2, attl2, attr2)
    acc2, s2 = _edge(al2.reshape(N_NODES), ar2.reshape(N_NODES),
                     amax2, src, dst, xl2, zrows, z1d)
    return _tc_out(acc2, s2.reshape(NW, N_PAD), Wp1, bp1, Wp2, bp2)


# speedup vs baseline: 23.2487x; 1.4870x over previous
import jax
import jax.numpy as jnp
from jax import lax
from jax.experimental import pallas as pl
from jax.experimental.pallas import tpu as pltpu
from jax.experimental.pallas import tpu_sc as plsc

N_NODES = 10000
N_EDGES = 320000
D = 128
NEG_SLOPE = 0.2

NC = 2
NS = 16
NW = NC * NS
NB = NS
EPB = N_EDGES // NB
HALF = 5120
NH = NC * HALF
RPS = HALF // NS
CH = 80
SEG = 2000
CAP = EPB + CH
SHIFT = 14
SENT = HALF << SHIFT
L = 16

_f32 = jnp.float32


def _proj(xl, attl_ref, attr_ref):
    dn = (((1,), (0,)), ((), ()))
    al = lax.dot_general(xl, attl_ref[...], dn,
                         preferred_element_type=_f32)
    ar = lax.dot_general(xl, attr_ref[...], dn,
                         preferred_element_type=_f32)
    amax = jnp.broadcast_to(jnp.max(al), (1, D))
    return al, ar, amax


def _tc_first_body(x_ref, w_ref, attl_ref, attr_ref,
                   xl_ref, al_ref, ar_ref, amax_ref):
    xl = lax.dot_general(x_ref[...], w_ref[...], (((1,), (1,)), ((), ())),
                         preferred_element_type=_f32)
    xl_ref[...] = xl
    al, ar, amax = _proj(xl, attl_ref, attr_ref)
    al_ref[...] = al
    ar_ref[...] = ar
    amax_ref[...] = amax


def _combine(acc_ref, s_ref):
    s = lax.dot_general(s_ref[...], jnp.ones((NS, 1), _f32),
                        (((0,), (0,)), ((), ())),
                        preferred_element_type=_f32)[:N_NODES]
    acc = acc_ref[...][:N_NODES]
    safe = jnp.where(s > 0, s, 1.0)
    h = jnp.where(s > 0, acc / safe, 0.0)
    return jnp.maximum(h, 0.0)


def _tc_mid_body(acc_ref, s_ref, w_ref, attl_ref, attr_ref,
                 xl_ref, al_ref, ar_ref, amax_ref):
    h = _combine(acc_ref, s_ref)
    xl = lax.dot_general(h, w_ref[...], (((1,), (1,)), ((), ())),
                         preferred_element_type=_f32)
    xl_ref[...] = xl
    al, ar, amax = _proj(xl, attl_ref, attr_ref)
    al_ref[...] = al
    ar_ref[...] = ar
    amax_ref[...] = amax


def _tc_out_body(acc_ref, s_ref, wp1_ref, bp1_ref, wp2_ref, bp2_ref, out_ref):
    h = _combine(acc_ref, s_ref)
    t = lax.dot_general(h, wp1_ref[...], (((1,), (1,)), ((), ())),
                        preferred_element_type=_f32) + bp1_ref[...][None, :]
    out_ref[...] = lax.dot_general(t, wp2_ref[...], (((1,), (1,)), ((), ())),
                                   preferred_element_type=_f32) + bp2_ref[...][None, :]


_lin_out = (jax.ShapeDtypeStruct((N_NODES, D), _f32),
            jax.ShapeDtypeStruct((N_NODES, 1), _f32),
            jax.ShapeDtypeStruct((N_NODES, 1), _f32),
            jax.ShapeDtypeStruct((1, D), _f32))

_tc_first = pl.pallas_call(_tc_first_body, out_shape=_lin_out)
_tc_mid = pl.pallas_call(_tc_mid_body, out_shape=_lin_out)
_tc_out = pl.pallas_call(
    _tc_out_body, out_shape=jax.ShapeDtypeStruct((N_NODES, D), _f32))


def _leaky(z):
    return jnp.where(z > 0, z, NEG_SLOPE * z)


def _edge_body(al_hbm, ar_hbm, amax_hbm, src_hbm, dst_hbm, xl_hbm,
               zrows_hbm, z1d_hbm, zsent_hbm,
               acc_out, s_out,
               al_v, ar_v, amax_v, s_v, seg_src, seg_dst, code_v,
               sidx2, didx2, wgt2, rows2,
               acc_sh, sem0, sem1):
    c = lax.axis_index("c")
    sid = lax.axis_index("s")
    w = sid * NC + c
    eb = sid * EPB
    lo = c * HALF

    pltpu.sync_copy(zrows_hbm, acc_sh.at[pl.ds(sid * RPS, RPS)])

    @pl.when(sid == 0)
    def _zero_tail():
        pltpu.sync_copy(zrows_hbm.at[pl.ds(0, 8)], acc_sh.at[pl.ds(HALF, 8)])

    pltpu.sync_copy(al_hbm, al_v)
    pltpu.sync_copy(ar_hbm, ar_v)
    pltpu.sync_copy(amax_hbm, amax_v)
    pltpu.sync_copy(z1d_hbm, s_v)
    pltpu.sync_copy(zsent_hbm, code_v)
    plsc.subcore_barrier()
    amax = amax_v[0, pl.ds(0, L)]

    def abody(k, pos):
        sb = eb + k * SEG
        pltpu.sync_copy(src_hbm.at[pl.ds(sb, SEG)], seg_src)
        pltpu.sync_copy(dst_hbm.at[pl.ds(sb, SEG)], seg_dst)

        def vbody(i, pos):
            off = pl.multiple_of(i * L, L)
            s16 = seg_src[pl.ds(off, L)]
            d16 = seg_dst[pl.ds(off, L)]
            als = plsc.load_gather(al_v, [s16])
            ard = plsc.load_gather(ar_v, [d16])
            a = _leaky(als + ard)
            md = _leaky(amax + ard)
            wv = jnp.exp(a - md)
            owned = (d16 >= lo) & (d16 < lo + HALF)
            ldst = d16 - lo
            plsc.addupdate_scatter(s_v, [jnp.where(owned, ldst, 0)], wv,
                                   mask=owned)
            code = s16 | lax.shift_left(ldst, SHIFT)
            plsc.store_compressed(code_v.at[pl.ds(pos, L)], code, mask=owned)
            cnt = jnp.max(plsc.all_reduce_population_count(owned))
            return pos + cnt

        return lax.fori_loop(0, SEG // L, vbody, pos)

    kcnt = lax.fori_loop(0, EPB // SEG, abody, 0)

    nit = (kcnt + 2 * CH - 1) // (2 * CH)

    def bbody(k, carry):
        base0 = pl.multiple_of(k * (2 * CH), L)
        cps = []
        for r in range(2):
            for q in range(CH // L):
                codeq = code_v[pl.ds(base0 + r * CH + q * L, L)]
                sidx2[r, pl.ds(q * L, L)] = codeq & ((1 << SHIFT) - 1)
                didx2[r, pl.ds(q * L, L)] = lax.shift_right_logical(codeq, SHIFT)
            sem = sem0 if r == 0 else sem1
            cps.append(pltpu.async_copy(xl_hbm.at[sidx2.at[r]],
                                        rows2.at[r], sem))
        for r in range(2):
            for q in range(CH // L):
                s16 = sidx2[r, pl.ds(q * L, L)]
                ld16 = didx2[r, pl.ds(q * L, L)]
                als = plsc.load_gather(al_v, [s16])
                g16 = jnp.minimum(ld16 + lo, N_NODES - 1)
                ard = plsc.load_gather(ar_v, [g16])
                wgt2[r, pl.ds(q * L, L)] = jnp.exp(_leaky(als + ard)
                                                   - _leaky(amax + ard))
            cps[r].wait()

            def sbody(j, inner):
                wj = plsc.load_gather(wgt2.at[r], [lax.broadcast(j, (L,))])
                for q in range(D // L):
                    sl = pl.ds(q * L, L)
                    rows2[r, j, sl] = rows2[r, j, sl] * wj
                return inner

            lax.fori_loop(0, CH, sbody, 0)
            pltpu.sync_copy(rows2.at[r], acc_sh.at[didx2.at[r]], add=True)
        return carry

    lax.fori_loop(0, nit, bbody, 0)

    plsc.subcore_barrier()
    pltpu.sync_copy(acc_sh.at[pl.ds(sid * RPS, RPS)],
                    acc_out.at[pl.ds(c * HALF + sid * RPS, RPS)])
    pltpu.sync_copy(s_v, s_out.at[pl.ds(w * HALF, HALF)])


_edge = pl.kernel(
    _edge_body,
    out_type=[jax.ShapeDtypeStruct((NH, D), _f32),
              jax.ShapeDtypeStruct((NW * HALF,), _f32)],
    mesh=plsc.VectorSubcoreMesh(core_axis_name="c", subcore_axis_name="s",
                                num_cores=NC, num_subcores=NS),
    scratch_types=[
        pltpu.VMEM((N_NODES,), _f32),
        pltpu.VMEM((N_NODES,), _f32),
        pltpu.VMEM((1, D), _f32),
        pltpu.VMEM((HALF,), _f32),
        pltpu.VMEM((SEG,), jnp.int32),
        pltpu.VMEM((SEG,), jnp.int32),
        pltpu.VMEM((CAP,), jnp.int32),
        pltpu.VMEM((2, CH), jnp.int32),
        pltpu.VMEM((2, CH), jnp.int32),
        pltpu.VMEM((2, CH), _f32),
        pltpu.VMEM((2, CH, D), _f32),
        pltpu.VMEM_SHARED((HALF + 8, D), _f32),
        pltpu.SemaphoreType.DMA,
        pltpu.SemaphoreType.DMA,
    ],
    compiler_params=pltpu.CompilerParams(needs_layout_passes=False),
)


def kernel(x, edge_index, W1, att_l1, att_r1, W2, att_l2, att_r2,
           Wp1, bp1, Wp2, bp2):
    src = edge_index[0].astype(jnp.int32)
    dst = edge_index[1].astype(jnp.int32)
    attl1 = att_l1.reshape(D, 1)
    attr1 = att_r1.reshape(D, 1)
    attl2 = att_l2.reshape(D, 1)
    attr2 = att_r2.reshape(D, 1)
    zrows = jnp.zeros((RPS, D), _f32)
    z1d = jnp.zeros((HALF,), _f32)
    zsent = jnp.full((CAP,), SENT, jnp.int32)

    xl1, al1, ar1, amax1 = _tc_first(x, W1, attl1, attr1)
    acc1, s1 = _edge(al1.reshape(N_NODES), ar1.reshape(N_NODES),
                     amax1, src, dst, xl1, zrows, z1d, zsent)
    xl2, al2, ar2, amax2 = _tc_mid(acc1, s1.reshape(NS, NH), W2, attl2, attr2)
    acc2, s2 = _edge(al2.reshape(N_NODES), ar2.reshape(N_NODES),
                     amax2, src, dst, xl2, zrows, z1d, zsent)
    return _tc_out(acc2, s2.reshape(NS, NH), Wp1, bp1, Wp2, bp2)
